# baseline (speedup 1.0000x reference)
import functools

import jax
import jax.numpy as jnp
from jax import lax
from jax.experimental import pallas as pl
from jax.experimental.pallas import tpu as pltpu
from jax.experimental.pallas import tpu_sc as plsc

_VOCAB = 1000000
_EMBED = 64
_NCLASS = 100
_B = 16384
_L = 200

_NC, _NS = 2, 16
_NW = _NC * _NS
_ROWS_W = _B // _NW
_IDX_W = _ROWS_W * _L
_CHUNK = 128
_NCHUNK_W = _IDX_W // _CHUNK
_STAGE = 40
_NOUT = _NCHUNK_W // _STAGE
_NBUF = 6
_GLEAD = 3
_ACC_ROWS = _NS * _ROWS_W

_MAGIC = 20972
_SHIFT = 19

_THALF = 512
_TGRID = (_VOCAB + 2 * _THALF - 1) // (2 * _THALF)
_TROWS = _TGRID * _THALF


def _tc_transpose(embed_t):
  def body(xl_ref, xr_ref, o_ref):
    o_ref[:, 0:_EMBED] = xl_ref[...].T
    o_ref[:, _EMBED:2 * _EMBED] = xr_ref[...].T

  return pl.pallas_call(
      body,
      grid=(_TGRID,),
      in_specs=[
          pl.BlockSpec((_EMBED, _THALF), lambda i: (0, 2 * i)),
          pl.BlockSpec((_EMBED, _THALF), lambda i: (0, 2 * i + 1)),
      ],
      out_specs=pl.BlockSpec((_THALF, 2 * _EMBED), lambda i: (i, 0)),
      out_shape=jax.ShapeDtypeStruct((_TROWS, 2 * _EMBED), jnp.float32),
  )(embed_t, embed_t)


def _sc_embedbag(idx3d, embed, zblock):
  mesh = plsc.VectorSubcoreMesh(core_axis_name="c", subcore_axis_name="s")

  @functools.partial(
      pl.kernel,
      out_type=jax.ShapeDtypeStruct((_B, _EMBED), jnp.float32),
      mesh=mesh,
      scratch_types=[
          pltpu.VMEM((_STAGE, _CHUNK), jnp.int32),
          [pltpu.VMEM((_CHUNK, _EMBED), jnp.float32) for _ in range(_NBUF)],
          [pltpu.VMEM((_CHUNK,), jnp.int32) for _ in range(_NBUF)],
          [pltpu.VMEM((_CHUNK,), jnp.int32) for _ in range(_NBUF)],
          [pltpu.SemaphoreType.DMA for _ in range(_NBUF)],
          [pltpu.SemaphoreType.DMA for _ in range(_NBUF)],
          pltpu.VMEM_SHARED((_ACC_ROWS, _EMBED), jnp.float32),
      ],
      compiler_params=pltpu.CompilerParams(use_tc_tiling_on_sc=False),
  )
  def k(idx_hbm, embed_hbm, z_hbm, y_hbm, idx_stage, rows, segs, didx, gsems,
        ssems, acc):
    c = lax.axis_index("c")
    s = lax.axis_index("s")
    w = c * _NS + s
    lane = lax.iota(jnp.int32, 16)
    s_off = s * _ROWS_W

    for t in range(_ROWS_W // _CHUNK):
      pltpu.sync_copy(z_hbm, acc.at[pl.ds(s_off + t * _CHUNK, _CHUNK)])

    def outer(o, carry):
      pltpu.sync_copy(idx_hbm.at[w * _NOUT + o], idx_stage)
      p_base = (o * _STAGE) * _CHUNK

      def fire_gather(jj):
        bb = jj % _NBUF
        for kk in range(8):
          v = idx_stage[jj, pl.ds(kk * 16, 16)]
          didx[bb][pl.ds(kk * 16, 16)] = (
              (v & -1024) + ((v & 511) << 1) + ((v >> 9) & 1))
        pltpu.async_copy(embed_hbm.at[didx[bb]], rows[bb], gsems[bb])

      for j in range(_GLEAD):
        fire_gather(j)
      for j in range(_STAGE):
        b = j % _NBUF
        pltpu.make_async_copy(embed_hbm.at[didx[b]], rows[b], gsems[b]).wait()
        for kk in range(8):
          p = lane + (p_base + j * _CHUNK + kk * 16)
          q = ((p >> 3) * _MAGIC) >> _SHIFT
          segs[b][pl.ds(kk * 16, 16)] = q + s_off
        pltpu.async_copy(rows[b], acc.at[segs[b]], ssems[b], add=True)
        jn = j + _GLEAD
        if jn < _STAGE:
          bn = jn % _NBUF
          if jn >= _NBUF:
            pltpu.make_async_copy(rows[bn], acc.at[segs[bn]], ssems[bn]).wait()
          fire_gather(jn)
      for m in range(_STAGE - _NBUF, _STAGE):
        bm = m % _NBUF
        pltpu.make_async_copy(rows[bm], acc.at[segs[bm]], ssems[bm]).wait()
      return carry

    lax.fori_loop(0, _NOUT, outer, 0)

    for t in range(_ROWS_W // _CHUNK):
      pltpu.sync_copy(acc.at[pl.ds(s_off + t * _CHUNK, _CHUNK)], rows[0])
      pltpu.sync_copy(rows[0], y_hbm.at[pl.ds(w * _ROWS_W + t * _CHUNK, _CHUNK)])

  return k(idx3d, embed, zblock)


def _tc_linear(y, w_mat, b2):
  bm = 2048

  def body(y_ref, w_ref, b_ref, o_ref):
    o_ref[...] = lax.dot_general(
        y_ref[...], w_ref[...], (((1,), (1,)), ((), ())),
        preferred_element_type=jnp.float32) + b_ref[...]

  return pl.pallas_call(
      body,
      grid=(_B // bm,),
      in_specs=[
          pl.BlockSpec((bm, _EMBED), lambda i: (i, 0)),
          pl.BlockSpec((_NCLASS, _EMBED), lambda i: (0, 0)),
          pl.BlockSpec((1, _NCLASS), lambda i: (0, 0)),
      ],
      out_specs=pl.BlockSpec((bm, _NCLASS), lambda i: (i, 0)),
      out_shape=jax.ShapeDtypeStruct((_B, _NCLASS), jnp.float32),
  )(y, w_mat, b2)


def kernel(input, embed, W, b):
  table = _tc_transpose(embed.T).reshape(2 * _TROWS, _EMBED)
  idx = input.astype(jnp.int32).reshape(_NW * _NOUT, _STAGE, _CHUNK)
  z = jnp.zeros((_CHUNK, _EMBED), jnp.float32)
  y = _sc_embedbag(idx, table, z)
  return _tc_linear(y, W, b.reshape(1, _NCLASS))


# speedup vs baseline: 1.3716x; 1.3716x over previous
import functools

import jax
import jax.numpy as jnp
from jax import lax
from jax.experimental import pallas as pl
from jax.experimental.pallas import tpu as pltpu
from jax.experimental.pallas import tpu_sc as plsc

_VOCAB = 1000000
_EMBED = 64
_NCLASS = 100
_B = 16384
_L = 200

_NC, _NS = 2, 16
_NW = _NC * _NS
_ROWS_W = _B // _NW
_IDX_W = _ROWS_W * _L
_CHUNK = 128
_NCHUNK_W = _IDX_W // _CHUNK
_STAGE = 40
_NOUT = _NCHUNK_W // _STAGE
_NBUF = 6
_GLEAD = 3
_ACC_ROWS = _NS * _ROWS_W

_MAGIC = 20972
_SHIFT = 19

_THALF = 2048
_TLOG = 11
_TGRID = (_VOCAB + 2 * _THALF - 1) // (2 * _THALF)
_TROWS = _TGRID * _THALF
_THBLK = _VOCAB // _THALF


def _tc_transpose(embed_t):
  def body(xl_ref, xr_ref, o_ref):
    o_ref[:, 0:_EMBED] = xl_ref[...].T
    o_ref[:, _EMBED:2 * _EMBED] = xr_ref[...].T

  return pl.pallas_call(
      body,
      grid=(_TGRID,),
      in_specs=[
          pl.BlockSpec((_EMBED, _THALF), lambda i: (0, 2 * i)),
          pl.BlockSpec((_EMBED, _THALF),
                       lambda i: (0, jnp.minimum(2 * i + 1, _THBLK))),
      ],
      out_specs=pl.BlockSpec((_THALF, 2 * _EMBED), lambda i: (i, 0)),
      out_shape=jax.ShapeDtypeStruct((_TROWS, 2 * _EMBED), jnp.float32),
  )(embed_t, embed_t)


def _sc_embedbag(idx3d, embed, zblock):
  mesh = plsc.VectorSubcoreMesh(core_axis_name="c", subcore_axis_name="s")

  @functools.partial(
      pl.kernel,
      out_type=jax.ShapeDtypeStruct((_B, _EMBED), jnp.float32),
      mesh=mesh,
      scratch_types=[
          pltpu.VMEM((_STAGE, _CHUNK), jnp.int32),
          [pltpu.VMEM((_CHUNK, _EMBED), jnp.float32) for _ in range(_NBUF)],
          [pltpu.VMEM((_CHUNK,), jnp.int32) for _ in range(_NBUF)],
          [pltpu.VMEM((_CHUNK,), jnp.int32) for _ in range(_NBUF)],
          [pltpu.SemaphoreType.DMA for _ in range(_NBUF)],
          [pltpu.SemaphoreType.DMA for _ in range(_NBUF)],
          pltpu.VMEM_SHARED((_ACC_ROWS, _EMBED), jnp.float32),
      ],
      compiler_params=pltpu.CompilerParams(use_tc_tiling_on_sc=False),
  )
  def k(idx_hbm, embed_hbm, z_hbm, y_hbm, idx_stage, rows, segs, didx, gsems,
        ssems, acc):
    c = lax.axis_index("c")
    s = lax.axis_index("s")
    w = c * _NS + s
    lane = lax.iota(jnp.int32, 16)
    s_off = s * _ROWS_W

    for t in range(_ROWS_W // _CHUNK):
      pltpu.sync_copy(z_hbm, acc.at[pl.ds(s_off + t * _CHUNK, _CHUNK)])

    def outer(o, carry):
      pltpu.sync_copy(idx_hbm.at[w * _NOUT + o], idx_stage)
      p_base = (o * _STAGE) * _CHUNK

      def fire_gather(jj):
        bb = jj % _NBUF
        for kk in range(8):
          v = idx_stage[jj, pl.ds(kk * 16, 16)]
          didx[bb][pl.ds(kk * 16, 16)] = (
              (v & (-2 * _THALF)) + ((v & (_THALF - 1)) << 1)
              + ((v >> _TLOG) & 1))
        pltpu.async_copy(embed_hbm.at[didx[bb]], rows[bb], gsems[bb])

      for j in range(_GLEAD):
        fire_gather(j)
      for j in range(_STAGE):
        b = j % _NBUF
        pltpu.make_async_copy(embed_hbm.at[didx[b]], rows[b], gsems[b]).wait()
        for kk in range(8):
          p = lane + (p_base + j * _CHUNK + kk * 16)
          q = ((p >> 3) * _MAGIC) >> _SHIFT
          segs[b][pl.ds(kk * 16, 16)] = q + s_off
        pltpu.async_copy(rows[b], acc.at[segs[b]], ssems[b], add=True)
        jn = j + _GLEAD
        if jn < _STAGE:
          bn = jn % _NBUF
          if jn >= _NBUF:
            pltpu.make_async_copy(rows[bn], acc.at[segs[bn]], ssems[bn]).wait()
          fire_gather(jn)
      for m in range(_STAGE - _NBUF, _STAGE):
        bm = m % _NBUF
        pltpu.make_async_copy(rows[bm], acc.at[segs[bm]], ssems[bm]).wait()
      return carry

    lax.fori_loop(0, _NOUT, outer, 0)

    for t in range(_ROWS_W // _CHUNK):
      pltpu.sync_copy(acc.at[pl.ds(s_off + t * _CHUNK, _CHUNK)], rows[0])
      pltpu.sync_copy(rows[0], y_hbm.at[pl.ds(w * _ROWS_W + t * _CHUNK, _CHUNK)])

  return k(idx3d, embed, zblock)


def _tc_linear(y, w_mat, b2):
  bm = 2048

  def body(y_ref, w_ref, b_ref, o_ref):
    o_ref[...] = lax.dot_general(
        y_ref[...], w_ref[...], (((1,), (1,)), ((), ())),
        preferred_element_type=jnp.float32) + b_ref[...]

  return pl.pallas_call(
      body,
      grid=(_B // bm,),
      in_specs=[
          pl.BlockSpec((bm, _EMBED), lambda i: (i, 0)),
          pl.BlockSpec((_NCLASS, _EMBED), lambda i: (0, 0)),
          pl.BlockSpec((1, _NCLASS), lambda i: (0, 0)),
      ],
      out_specs=pl.BlockSpec((bm, _NCLASS), lambda i: (i, 0)),
      out_shape=jax.ShapeDtypeStruct((_B, _NCLASS), jnp.float32),
  )(y, w_mat, b2)


def kernel(input, embed, W, b):
  table = _tc_transpose(embed.T).reshape(2 * _TROWS, _EMBED)
  idx = input.astype(jnp.int32).reshape(_NW * _NOUT, _STAGE, _CHUNK)
  z = jnp.zeros((_CHUNK, _EMBED), jnp.float32)
  y = _sc_embedbag(idx, table, z)
  return _tc_linear(y, W, b.reshape(1, _NCLASS))


# speedup vs baseline: 1.4784x; 1.0778x over previous
import functools

import jax
import jax.numpy as jnp
from jax import lax
from jax.experimental import pallas as pl
from jax.experimental.pallas import tpu as pltpu
from jax.experimental.pallas import tpu_sc as plsc

_VOCAB = 1000000
_EMBED = 64
_NCLASS = 100
_B = 16384
_L = 200

_NC, _NS = 2, 16
_NW = _NC * _NS
_ROWS_W = _B // _NW
_IDX_W = _ROWS_W * _L
_CHUNK = 128
_NCHUNK_W = _IDX_W // _CHUNK
_STAGE = 40
_NOUT = _NCHUNK_W // _STAGE
_NBUF = 6
_GLEAD = 3
_ACC_ROWS = _NS * _ROWS_W

_MAGIC = 20972
_SHIFT = 19

_THALF = 4096
_TLOG = 12
_TGRID = (_VOCAB + 2 * _THALF - 1) // (2 * _THALF)
_TROWS = _TGRID * _THALF
_THBLK = _VOCAB // _THALF


def _tc_transpose(embed_t):
  def body(xl_ref, xr_ref, o_ref):
    o_ref[:, 0:_EMBED] = xl_ref[...].T
    o_ref[:, _EMBED:2 * _EMBED] = xr_ref[...].T

  return pl.pallas_call(
      body,
      grid=(_TGRID,),
      in_specs=[
          pl.BlockSpec((_EMBED, _THALF), lambda i: (0, 2 * i)),
          pl.BlockSpec((_EMBED, _THALF),
                       lambda i: (0, jnp.minimum(2 * i + 1, _THBLK))),
      ],
      out_specs=pl.BlockSpec((_THALF, 2 * _EMBED), lambda i: (i, 0)),
      out_shape=jax.ShapeDtypeStruct((_TROWS, 2 * _EMBED), jnp.float32),
  )(embed_t, embed_t)


def _sc_embedbag(idx3d, embed, zblock):
  mesh = plsc.VectorSubcoreMesh(core_axis_name="c", subcore_axis_name="s")

  @functools.partial(
      pl.kernel,
      out_type=jax.ShapeDtypeStruct((_B, _EMBED), jnp.float32),
      mesh=mesh,
      scratch_types=[
          pltpu.VMEM((_STAGE, _CHUNK), jnp.int32),
          [pltpu.VMEM((_CHUNK, _EMBED), jnp.float32) for _ in range(_NBUF)],
          [pltpu.VMEM((_CHUNK,), jnp.int32) for _ in range(_NBUF)],
          [pltpu.VMEM((_CHUNK,), jnp.int32) for _ in range(_NBUF)],
          [pltpu.SemaphoreType.DMA for _ in range(_NBUF)],
          [pltpu.SemaphoreType.DMA for _ in range(_NBUF)],
          pltpu.VMEM_SHARED((_ACC_ROWS, _EMBED), jnp.float32),
      ],
      compiler_params=pltpu.CompilerParams(use_tc_tiling_on_sc=False),
  )
  def k(idx_hbm, embed_hbm, z_hbm, y_hbm, idx_stage, rows, segs, didx, gsems,
        ssems, acc):
    c = lax.axis_index("c")
    s = lax.axis_index("s")
    w = c * _NS + s
    lane = lax.iota(jnp.int32, 16)
    s_off = s * _ROWS_W

    for t in range(_ROWS_W // _CHUNK):
      pltpu.sync_copy(z_hbm, acc.at[pl.ds(s_off + t * _CHUNK, _CHUNK)])

    def outer(o, carry):
      pltpu.sync_copy(idx_hbm.at[w * _NOUT + o], idx_stage)
      p_base = (o * _STAGE) * _CHUNK

      def fire_gather(jj):
        bb = jj % _NBUF
        for kk in range(8):
          v = idx_stage[jj, pl.ds(kk * 16, 16)]
          didx[bb][pl.ds(kk * 16, 16)] = (
              (v & (-2 * _THALF)) + ((v & (_THALF - 1)) << 1)
              + ((v >> _TLOG) & 1))
        pltpu.async_copy(embed_hbm.at[didx[bb]], rows[bb], gsems[bb])

      for j in range(_GLEAD):
        fire_gather(j)
      for j in range(_STAGE):
        b = j % _NBUF
        pltpu.make_async_copy(embed_hbm.at[didx[b]], rows[b], gsems[b]).wait()
        for kk in range(8):
          p = lane + (p_base + j * _CHUNK + kk * 16)
          q = ((p >> 3) * _MAGIC) >> _SHIFT
          segs[b][pl.ds(kk * 16, 16)] = q + s_off
        pltpu.async_copy(rows[b], acc.at[segs[b]], ssems[b], add=True)
        jn = j + _GLEAD
        if jn < _STAGE:
          bn = jn % _NBUF
          if jn >= _NBUF:
            pltpu.make_async_copy(rows[bn], acc.at[segs[bn]], ssems[bn]).wait()
          fire_gather(jn)
      for m in range(_STAGE - _NBUF, _STAGE):
        bm = m % _NBUF
        pltpu.make_async_copy(rows[bm], acc.at[segs[bm]], ssems[bm]).wait()
      return carry

    lax.fori_loop(0, _NOUT, outer, 0)

    for t in range(_ROWS_W // _CHUNK):
      pltpu.sync_copy(acc.at[pl.ds(s_off + t * _CHUNK, _CHUNK)], rows[0])
      pltpu.sync_copy(rows[0], y_hbm.at[pl.ds(w * _ROWS_W + t * _CHUNK, _CHUNK)])

  return k(idx3d, embed, zblock)


def _tc_linear(y, w_mat, b2):
  bm = 2048

  def body(y_ref, w_ref, b_ref, o_ref):
    o_ref[...] = lax.dot_general(
        y_ref[...], w_ref[...], (((1,), (1,)), ((), ())),
        preferred_element_type=jnp.float32) + b_ref[...]

  return pl.pallas_call(
      body,
      grid=(_B // bm,),
      in_specs=[
          pl.BlockSpec((bm, _EMBED), lambda i: (i, 0)),
          pl.BlockSpec((_NCLASS, _EMBED), lambda i: (0, 0)),
          pl.BlockSpec((1, _NCLASS), lambda i: (0, 0)),
      ],
      out_specs=pl.BlockSpec((bm, _NCLASS), lambda i: (i, 0)),
      out_shape=jax.ShapeDtypeStruct((_B, _NCLASS), jnp.float32),
  )(y, w_mat, b2)


def kernel(input, embed, W, b):
  table = _tc_transpose(embed.T).reshape(2 * _TROWS, _EMBED)
  idx = input.astype(jnp.int32).reshape(_NW * _NOUT, _STAGE, _CHUNK)
  z = jnp.zeros((_CHUNK, _EMBED), jnp.float32)
  y = _sc_embedbag(idx, table, z)
  return _tc_linear(y, W, b.reshape(1, _NCLASS))


# speedup vs baseline: 1.5277x; 1.0334x over previous
import functools

import jax
import jax.numpy as jnp
from jax import lax
from jax.experimental import pallas as pl
from jax.experimental.pallas import tpu as pltpu
from jax.experimental.pallas import tpu_sc as plsc

_VOCAB = 1000000
_EMBED = 64
_NCLASS = 100
_B = 16384
_L = 200

_NC, _NS = 2, 16
_NW = _NC * _NS
_ROWS_W = _B // _NW
_IDX_W = _ROWS_W * _L
_CHUNK = 128
_NCHUNK_W = _IDX_W // _CHUNK
_STAGE = 40
_NOUT = _NCHUNK_W // _STAGE
_NBUF = 6
_GLEAD = 3
_ACC_ROWS = _NS * _ROWS_W

_MAGIC = 20972
_SHIFT = 19

_THALF = 4096
_TLOG = 12
_TGRID = (_VOCAB + 2 * _THALF - 1) // (2 * _THALF)
_TROWS = _TGRID * _THALF
_THBLK = _VOCAB // _THALF


def _tc_transpose(embed_t):
  def body(xl_ref, xr_ref, o_ref):
    o_ref[:, 0:_EMBED] = xl_ref[...].T
    o_ref[:, _EMBED:2 * _EMBED] = xr_ref[...].T

  return pl.pallas_call(
      body,
      grid=(_TGRID,),
      in_specs=[
          pl.BlockSpec((_EMBED, _THALF), lambda i: (0, 2 * i)),
          pl.BlockSpec((_EMBED, _THALF),
                       lambda i: (0, jnp.minimum(2 * i + 1, _THBLK))),
      ],
      out_specs=pl.BlockSpec((_THALF, 2 * _EMBED), lambda i: (i, 0)),
      out_shape=jax.ShapeDtypeStruct((_TROWS, 2 * _EMBED), jnp.float32),
  )(embed_t, embed_t)


def _sc_embedbag(idx3d, embed, zblock):
  mesh = plsc.VectorSubcoreMesh(core_axis_name="c", subcore_axis_name="s")

  @functools.partial(
      pl.kernel,
      out_type=jax.ShapeDtypeStruct((_B, _EMBED), jnp.float32),
      mesh=mesh,
      scratch_types=[
          pltpu.VMEM((_STAGE, _CHUNK), jnp.int32),
          [pltpu.VMEM((_CHUNK, _EMBED), jnp.float32) for _ in range(_NBUF)],
          [pltpu.VMEM((_CHUNK,), jnp.int32) for _ in range(_NBUF)],
          [pltpu.VMEM((_CHUNK,), jnp.int32) for _ in range(_NBUF)],
          [pltpu.SemaphoreType.DMA for _ in range(_NBUF)],
          [pltpu.SemaphoreType.DMA for _ in range(_NBUF)],
          pltpu.VMEM_SHARED((_ACC_ROWS, _EMBED), jnp.float32),
      ],
      compiler_params=pltpu.CompilerParams(use_tc_tiling_on_sc=False),
  )
  def k(idx_hbm, embed_hbm, z_hbm, y_hbm, idx_stage, rows, segs, didx, gsems,
        ssems, acc):
    c = lax.axis_index("c")
    s = lax.axis_index("s")
    w = c * _NS + s
    lane = lax.iota(jnp.int32, 16)
    s_off = s * _ROWS_W

    for t in range(_ROWS_W // _CHUNK):
      pltpu.sync_copy(z_hbm, acc.at[pl.ds(s_off + t * _CHUNK, _CHUNK)])

    def outer(o, carry):
      pltpu.sync_copy(idx_hbm.at[w * _NOUT + o], idx_stage)
      p_base = (o * _STAGE) * _CHUNK

      def fire_gather(jj):
        bb = jj % _NBUF
        for kk in range(8):
          v = idx_stage[jj, pl.ds(kk * 16, 16)]
          didx[bb][pl.ds(kk * 16, 16)] = (
              (v & (-2 * _THALF)) + ((v & (_THALF - 1)) << 1)
              + ((v >> _TLOG) & 1))
        pltpu.async_copy(embed_hbm.at[didx[bb]], rows[bb], gsems[bb])

      for j in range(_GLEAD):
        fire_gather(j)
      for j in range(_STAGE):
        b = j % _NBUF
        pltpu.make_async_copy(embed_hbm.at[didx[b]], rows[b], gsems[b]).wait()
        for kk in range(8):
          p = lane + (p_base + j * _CHUNK + kk * 16)
          q = ((p >> 3) * _MAGIC) >> _SHIFT
          segs[b][pl.ds(kk * 16, 16)] = q + s_off
        if j >= 2:
          bp = (j - 2) % _NBUF
          pltpu.make_async_copy(rows[bp], acc.at[segs[bp]], ssems[bp]).wait()
        pltpu.async_copy(rows[b], acc.at[segs[b]], ssems[b], add=True)
        jn = j + _GLEAD
        if jn < _STAGE:
          fire_gather(jn)
      for m in range(_STAGE - 2, _STAGE):
        bm = m % _NBUF
        pltpu.make_async_copy(rows[bm], acc.at[segs[bm]], ssems[bm]).wait()
      return carry

    lax.fori_loop(0, _NOUT, outer, 0)

    for t in range(_ROWS_W // _CHUNK):
      pltpu.sync_copy(acc.at[pl.ds(s_off + t * _CHUNK, _CHUNK)], rows[0])
      pltpu.sync_copy(rows[0], y_hbm.at[pl.ds(w * _ROWS_W + t * _CHUNK, _CHUNK)])

  return k(idx3d, embed, zblock)


def _tc_linear(y, w_mat, b2):
  bm = 2048

  def body(y_ref, w_ref, b_ref, o_ref):
    o_ref[...] = lax.dot_general(
        y_ref[...], w_ref[...], (((1,), (1,)), ((), ())),
        preferred_element_type=jnp.float32) + b_ref[...]

  return pl.pallas_call(
      body,
      grid=(_B // bm,),
      in_specs=[
          pl.BlockSpec((bm, _EMBED), lambda i: (i, 0)),
          pl.BlockSpec((_NCLASS, _EMBED), lambda i: (0, 0)),
          pl.BlockSpec((1, _NCLASS), lambda i: (0, 0)),
      ],
      out_specs=pl.BlockSpec((bm, _NCLASS), lambda i: (i, 0)),
      out_shape=jax.ShapeDtypeStruct((_B, _NCLASS), jnp.float32),
  )(y, w_mat, b2)


def kernel(input, embed, W, b):
  table = _tc_transpose(embed.T).reshape(2 * _TROWS, _EMBED)
  idx = input.astype(jnp.int32).reshape(_NW * _NOUT, _STAGE, _CHUNK)
  z = jnp.zeros((_CHUNK, _EMBED), jnp.float32)
  y = _sc_embedbag(idx, table, z)
  return _tc_linear(y, W, b.reshape(1, _NCLASS))


# speedup vs baseline: 1.5400x; 1.0081x over previous
import functools

import jax
import jax.numpy as jnp
from jax import lax
from jax.experimental import pallas as pl
from jax.experimental.pallas import tpu as pltpu
from jax.experimental.pallas import tpu_sc as plsc

_VOCAB = 1000000
_EMBED = 64
_NCLASS = 100
_B = 16384
_L = 200

_NC, _NS = 2, 16
_NW = _NC * _NS
_ROWS_W = _B // _NW
_IDX_W = _ROWS_W * _L
_CHUNK = 128
_NCHUNK_W = _IDX_W // _CHUNK
_STAGE = 40
_NOUT = _NCHUNK_W // _STAGE
_NBUF = 8
_GLEAD = 5
_ACC_ROWS = _NS * _ROWS_W

_MAGIC = 20972
_SHIFT = 19

_THALF = 4096
_TLOG = 12
_TGRID = (_VOCAB + 2 * _THALF - 1) // (2 * _THALF)
_TROWS = _TGRID * _THALF
_THBLK = _VOCAB // _THALF


def _tc_transpose(embed_t):
  def body(xl_ref, xr_ref, o_ref):
    o_ref[:, 0:_EMBED] = xl_ref[...].T
    o_ref[:, _EMBED:2 * _EMBED] = xr_ref[...].T

  return pl.pallas_call(
      body,
      grid=(_TGRID,),
      in_specs=[
          pl.BlockSpec((_EMBED, _THALF), lambda i: (0, 2 * i)),
          pl.BlockSpec((_EMBED, _THALF),
                       lambda i: (0, jnp.minimum(2 * i + 1, _THBLK))),
      ],
      out_specs=pl.BlockSpec((_THALF, 2 * _EMBED), lambda i: (i, 0)),
      out_shape=jax.ShapeDtypeStruct((_TROWS, 2 * _EMBED), jnp.float32),
  )(embed_t, embed_t)


def _sc_embedbag(idx3d, embed, zblock):
  mesh = plsc.VectorSubcoreMesh(core_axis_name="c", subcore_axis_name="s")

  @functools.partial(
      pl.kernel,
      out_type=jax.ShapeDtypeStruct((_B, _EMBED), jnp.float32),
      mesh=mesh,
      scratch_types=[
          pltpu.VMEM((_STAGE, _CHUNK), jnp.int32),
          [pltpu.VMEM((_CHUNK, _EMBED), jnp.float32) for _ in range(_NBUF)],
          [pltpu.VMEM((_CHUNK,), jnp.int32) for _ in range(_NBUF)],
          [pltpu.VMEM((_CHUNK,), jnp.int32) for _ in range(_NBUF)],
          [pltpu.SemaphoreType.DMA for _ in range(_NBUF)],
          [pltpu.SemaphoreType.DMA for _ in range(_NBUF)],
          pltpu.VMEM_SHARED((_ACC_ROWS, _EMBED), jnp.float32),
      ],
      compiler_params=pltpu.CompilerParams(use_tc_tiling_on_sc=False),
  )
  def k(idx_hbm, embed_hbm, z_hbm, y_hbm, idx_stage, rows, segs, didx, gsems,
        ssems, acc):
    c = lax.axis_index("c")
    s = lax.axis_index("s")
    w = c * _NS + s
    lane = lax.iota(jnp.int32, 16)
    s_off = s * _ROWS_W

    for t in range(_ROWS_W // _CHUNK):
      pltpu.sync_copy(z_hbm, acc.at[pl.ds(s_off + t * _CHUNK, _CHUNK)])

    def outer(o, carry):
      pltpu.sync_copy(idx_hbm.at[w * _NOUT + o], idx_stage)
      p_base = (o * _STAGE) * _CHUNK

      def fire_gather(jj):
        bb = jj % _NBUF
        for kk in range(8):
          v = idx_stage[jj, pl.ds(kk * 16, 16)]
          didx[bb][pl.ds(kk * 16, 16)] = (
              (v & (-2 * _THALF)) + ((v & (_THALF - 1)) << 1)
              + ((v >> _TLOG) & 1))
        pltpu.async_copy(embed_hbm.at[didx[bb]], rows[bb], gsems[bb])

      for j in range(_GLEAD):
        fire_gather(j)
      for j in range(_STAGE):
        b = j % _NBUF
        pltpu.make_async_copy(embed_hbm.at[didx[b]], rows[b], gsems[b]).wait()
        for kk in range(8):
          p = lane + (p_base + j * _CHUNK + kk * 16)
          q = ((p >> 3) * _MAGIC) >> _SHIFT
          segs[b][pl.ds(kk * 16, 16)] = q + s_off
        if j >= 2:
          bp = (j - 2) % _NBUF
          pltpu.make_async_copy(rows[bp], acc.at[segs[bp]], ssems[bp]).wait()
        pltpu.async_copy(rows[b], acc.at[segs[b]], ssems[b], add=True)
        jn = j + _GLEAD
        if jn < _STAGE:
          fire_gather(jn)
      for m in range(_STAGE - 2, _STAGE):
        bm = m % _NBUF
        pltpu.make_async_copy(rows[bm], acc.at[segs[bm]], ssems[bm]).wait()
      return carry

    lax.fori_loop(0, _NOUT, outer, 0)

    for t in range(_ROWS_W // _CHUNK):
      pltpu.sync_copy(acc.at[pl.ds(s_off + t * _CHUNK, _CHUNK)], rows[0])
      pltpu.sync_copy(rows[0], y_hbm.at[pl.ds(w * _ROWS_W + t * _CHUNK, _CHUNK)])

  return k(idx3d, embed, zblock)


def _tc_linear(y, w_mat, b2):
  bm = 2048

  def body(y_ref, w_ref, b_ref, o_ref):
    o_ref[...] = lax.dot_general(
        y_ref[...], w_ref[...], (((1,), (1,)), ((), ())),
        preferred_element_type=jnp.float32) + b_ref[...]

  return pl.pallas_call(
      body,
      grid=(_B // bm,),
      in_specs=[
          pl.BlockSpec((bm, _EMBED), lambda i: (i, 0)),
          pl.BlockSpec((_NCLASS, _EMBED), lambda i: (0, 0)),
          pl.BlockSpec((1, _NCLASS), lambda i: (0, 0)),
      ],
      out_specs=pl.BlockSpec((bm, _NCLASS), lambda i: (i, 0)),
      out_shape=jax.ShapeDtypeStruct((_B, _NCLASS), jnp.float32),
  )(y, w_mat, b2)


def kernel(input, embed, W, b):
  table = _tc_transpose(embed.T).reshape(2 * _TROWS, _EMBED)
  idx = input.astype(jnp.int32).reshape(_NW * _NOUT, _STAGE, _CHUNK)
  z = jnp.zeros((_CHUNK, _EMBED), jnp.float32)
  y = _sc_embedbag(idx, table, z)
  return _tc_linear(y, W, b.reshape(1, _NCLASS))
